# baseline (speedup 1.0000x reference)
import functools

import jax
import jax.numpy as jnp
from jax import lax
from jax.experimental import pallas as pl
from jax.experimental.pallas import tpu as pltpu
from jax.experimental.pallas import tpu_sc as plsc

_CHUNK = 128
_NBUF = 2


@functools.lru_cache(maxsize=None)
def _make_gather(n_rows, d):
    info = plsc.get_sparse_core_info()
    num_cores, num_subcores = info.num_cores, info.num_subcores
    num_workers = num_cores * num_subcores
    n_chunks = n_rows // _CHUNK
    per_w = n_chunks // num_workers
    assert per_w * num_workers == n_chunks and per_w % _NBUF == 0

    mesh = plsc.VectorSubcoreMesh(core_axis_name="c", subcore_axis_name="s")

    @functools.partial(
        pl.kernel,
        out_type=jax.ShapeDtypeStruct((n_rows, d), jnp.float32),
        mesh=mesh,
        scratch_types=[
            pltpu.VMEM((per_w, _CHUNK), jnp.int32),
            pltpu.VMEM((_NBUF, _CHUNK, d), jnp.float32),
        ]
        + [pltpu.SemaphoreType.DMA] * _NBUF,
        compiler_params=pltpu.CompilerParams(use_tc_tiling_on_sc=False),
    )
    def gather_kernel(table_hbm, idx_hbm, out_hbm, idx_v, rows_v, *sems):
        wid = lax.axis_index("s") * num_cores + lax.axis_index("c")
        chunk0 = wid * per_w
        pltpu.sync_copy(idx_hbm.at[wid], idx_v)
        for b in range(_NBUF):
            pltpu.make_async_copy(
                table_hbm.at[idx_v.at[b]], rows_v.at[b], sems[b]
            ).start()

        def outer(g, carry):
            for b in range(_NBUF):
                j = g * _NBUF + b
                pltpu.make_async_copy(
                    table_hbm.at[idx_v.at[j]], rows_v.at[b], sems[b]
                ).wait()
                pltpu.sync_copy(
                    rows_v.at[b],
                    out_hbm.at[pl.ds((chunk0 + j) * _CHUNK, _CHUNK)],
                )
                nxt = j + _NBUF

                @pl.when(nxt < per_w)
                def _():
                    pltpu.make_async_copy(
                        table_hbm.at[idx_v.at[nxt]], rows_v.at[b], sems[b]
                    ).start()

            return carry

        lax.fori_loop(0, per_w // _NBUF, outer, 0)

    return gather_kernel


def kernel(matrix, node_seq_id, G=0):
    batch, hist = node_seq_id.shape
    d = matrix.shape[1]
    n_rows = batch * hist
    info = plsc.get_sparse_core_info()
    num_workers = info.num_cores * info.num_subcores
    idx3d = node_seq_id.reshape(num_workers, -1, _CHUNK)
    out = _make_gather(n_rows, d)(matrix, idx3d)
    return out.reshape(batch, hist, d)


# speedup vs baseline: 1.0109x; 1.0109x over previous
import functools

import jax
import jax.numpy as jnp
from jax import lax
from jax.experimental import pallas as pl
from jax.experimental.pallas import tpu as pltpu
from jax.experimental.pallas import tpu_sc as plsc

_CHUNK = 128
_NBUF = 5


@functools.lru_cache(maxsize=None)
def _make_gather(n_rows, d):
    info = plsc.get_sparse_core_info()
    num_cores, num_subcores = info.num_cores, info.num_subcores
    num_workers = num_cores * num_subcores
    n_chunks = n_rows // _CHUNK
    per_w = n_chunks // num_workers
    assert per_w * num_workers == n_chunks and per_w % _NBUF == 0

    mesh = plsc.VectorSubcoreMesh(core_axis_name="c", subcore_axis_name="s")

    @functools.partial(
        pl.kernel,
        out_type=jax.ShapeDtypeStruct((n_rows, d), jnp.float32),
        mesh=mesh,
        scratch_types=[
            pltpu.VMEM((per_w, _CHUNK), jnp.int32),
            pltpu.VMEM((_NBUF, _CHUNK, d), jnp.float32),
        ]
        + [pltpu.SemaphoreType.DMA] * (2 * _NBUF),
        compiler_params=pltpu.CompilerParams(use_tc_tiling_on_sc=False),
    )
    def gather_kernel(table_hbm, idx_hbm, out_hbm, idx_v, rows_v, *sems):
        gsem = sems[:_NBUF]
        ssem = sems[_NBUF:]
        wid = lax.axis_index("s") * num_cores + lax.axis_index("c")
        chunk0 = wid * per_w
        pltpu.sync_copy(idx_hbm.at[wid], idx_v)
        for b in range(_NBUF):
            pltpu.make_async_copy(
                table_hbm.at[idx_v.at[b]], rows_v.at[b], gsem[b]
            ).start()

        def out_slice(j):
            return out_hbm.at[pl.ds((chunk0 + j) * _CHUNK, _CHUNK)]

        def outer(g, carry):
            for b in range(_NBUF):
                j = g * _NBUF + b
                pltpu.make_async_copy(
                    table_hbm.at[idx_v.at[j]], rows_v.at[b], gsem[b]
                ).wait()
                pltpu.make_async_copy(rows_v.at[b], out_slice(j), ssem[b]).start()
                bp = (b - 1) % _NBUF
                jp = j - 1 + _NBUF

                @pl.when((j >= 1) & (jp < per_w))
                def _():
                    pltpu.make_async_copy(
                        rows_v.at[bp], out_slice(jp - _NBUF), ssem[bp]
                    ).wait()
                    pltpu.make_async_copy(
                        table_hbm.at[idx_v.at[jp]], rows_v.at[bp], gsem[bp]
                    ).start()

            return carry

        lax.fori_loop(0, per_w // _NBUF, outer, 0)
        for b in range(_NBUF):
            j = per_w - _NBUF + b
            pltpu.make_async_copy(rows_v.at[b], out_slice(j), ssem[b]).wait()

    return gather_kernel


def kernel(matrix, node_seq_id, G=0):
    batch, hist = node_seq_id.shape
    d = matrix.shape[1]
    n_rows = batch * hist
    info = plsc.get_sparse_core_info()
    num_workers = info.num_cores * info.num_subcores
    idx3d = node_seq_id.reshape(num_workers, -1, _CHUNK)
    out = _make_gather(n_rows, d)(matrix, idx3d)
    return out.reshape(batch, hist, d)
